# baseline (speedup 1.0000x reference)
import jax
import jax.numpy as jnp
from jax.experimental import pallas as pl
from jax.experimental.pallas import tpu as pltpu

_N = 8192
_D = 64
_BM = 128
_S = 8
_NB = _N // _BM


def _fused_block(adj_hbm, h_ref, hblk_ref, wt_ref, b_ref, out_ref, buf, sem):
    i = pl.program_id(0)

    def start_copy(block, slot):
        pltpu.make_async_copy(
            adj_hbm.at[pl.ds(block * _BM, _BM), :], buf.at[slot], sem.at[slot]
        ).start()

    @pl.when(i == 0)
    def _prologue():
        for k in range(_S - 1):
            start_copy(k, k)

    nxt = i + _S - 1

    @pl.when(nxt < _NB)
    def _prefetch():
        start_copy(nxt, jax.lax.rem(nxt, _S))

    slot = jax.lax.rem(i, _S)
    pltpu.make_async_copy(
        adj_hbm.at[pl.ds(i * _BM, _BM), :], buf.at[slot], sem.at[slot]
    ).wait()

    adj = buf[slot]
    acc = jnp.dot(adj, h_ref[...], preferred_element_type=jnp.float32)
    rs = jnp.sum(adj, axis=1, keepdims=True)
    pooled = acc / rs
    cat = jnp.concatenate([hblk_ref[...], pooled], axis=1)
    out_ref[...] = (
        jnp.dot(cat, wt_ref[...], preferred_element_type=jnp.float32) + b_ref[...]
    )


@jax.jit
def kernel(h, adj, W_lg, b_lg):
    n, d = h.shape
    wt = W_lg.T
    b = b_lg.reshape(1, d)
    return pl.pallas_call(
        _fused_block,
        grid=(_NB,),
        in_specs=[
            pl.BlockSpec(memory_space=pl.ANY),
            pl.BlockSpec((n, d), lambda i: (0, 0)),
            pl.BlockSpec((_BM, d), lambda i: (i, 0)),
            pl.BlockSpec((2 * d, d), lambda i: (0, 0)),
            pl.BlockSpec((1, d), lambda i: (0, 0)),
        ],
        out_specs=pl.BlockSpec((_BM, d), lambda i: (i, 0)),
        out_shape=jax.ShapeDtypeStruct((n, d), jnp.float32),
        scratch_shapes=[
            pltpu.VMEM((_S, _BM, _N), jnp.float32),
            pltpu.SemaphoreType.DMA((_S,)),
        ],
    )(adj, h, h, wt, b)


# speedup vs baseline: 1.0204x; 1.0204x over previous
import jax
import jax.numpy as jnp
from jax.experimental import pallas as pl
from jax.experimental.pallas import tpu as pltpu

_N = 8192
_D = 64
_BM = 256
_S = 4
_NB = _N // _BM


def _fused_block(adj_hbm, h_hbm, hblk_ref, wt_ref, b_ref, out_ref, buf, sem, hbuf, hsem):
    i = pl.program_id(0)

    def start_copy(block, slot):
        pltpu.make_async_copy(
            adj_hbm.at[pl.ds(block * _BM, _BM), :], buf.at[slot], sem.at[slot]
        ).start()

    @pl.when(i == 0)
    def _prologue():
        pltpu.make_async_copy(h_hbm, hbuf, hsem).start()
        for k in range(_S - 1):
            start_copy(k, k)
        pltpu.make_async_copy(h_hbm, hbuf, hsem).wait()

    nxt = i + _S - 1

    @pl.when(nxt < _NB)
    def _prefetch():
        start_copy(nxt, jax.lax.rem(nxt, _S))

    slot = jax.lax.rem(i, _S)
    pltpu.make_async_copy(
        adj_hbm.at[pl.ds(i * _BM, _BM), :], buf.at[slot], sem.at[slot]
    ).wait()

    adj = buf[slot]
    acc = jnp.dot(adj, hbuf[...], preferred_element_type=jnp.float32)
    rs = jnp.sum(adj, axis=1, keepdims=True)
    pooled = acc / rs
    cat = jnp.concatenate([hblk_ref[...], pooled], axis=1)
    out_ref[...] = (
        jnp.dot(cat, wt_ref[...], preferred_element_type=jnp.float32) + b_ref[...]
    )


@jax.jit
def kernel(h, adj, W_lg, b_lg):
    n, d = h.shape
    wt = W_lg.T
    b = b_lg.reshape(1, d)
    return pl.pallas_call(
        _fused_block,
        grid=(_NB,),
        in_specs=[
            pl.BlockSpec(memory_space=pl.ANY),
            pl.BlockSpec(memory_space=pl.ANY),
            pl.BlockSpec((_BM, d), lambda i: (i, 0)),
            pl.BlockSpec((2 * d, d), lambda i: (0, 0)),
            pl.BlockSpec((1, d), lambda i: (0, 0)),
        ],
        out_specs=pl.BlockSpec((_BM, d), lambda i: (i, 0)),
        out_shape=jax.ShapeDtypeStruct((n, d), jnp.float32),
        scratch_shapes=[
            pltpu.VMEM((_S, _BM, _N), jnp.float32),
            pltpu.SemaphoreType.DMA((_S,)),
            pltpu.VMEM((_N, _D), jnp.float32),
            pltpu.SemaphoreType.DMA,
        ],
    )(adj, h, h, wt, b)


# speedup vs baseline: 1.0295x; 1.0089x over previous
import jax
import jax.numpy as jnp
from jax.experimental import pallas as pl
from jax.experimental.pallas import tpu as pltpu

_N = 8192
_D = 64
_BM = 256
_S = 4
_NB = _N // _BM


def _fused_block(adj_hbm, h_ref, hblk_ref, wt_ref, b_ref, out_ref, buf, sem):
    i = pl.program_id(0)

    def start_copy(block, slot):
        pltpu.make_async_copy(
            adj_hbm.at[pl.ds(block * _BM, _BM), :], buf.at[slot], sem.at[slot]
        ).start()

    @pl.when(i == 0)
    def _prologue():
        for k in range(_S - 1):
            start_copy(k, k)

    nxt = i + _S - 1

    @pl.when(nxt < _NB)
    def _prefetch():
        start_copy(nxt, jax.lax.rem(nxt, _S))

    slot = jax.lax.rem(i, _S)
    pltpu.make_async_copy(
        adj_hbm.at[pl.ds(i * _BM, _BM), :], buf.at[slot], sem.at[slot]
    ).wait()

    adj = buf[slot]
    acc = jnp.dot(adj, h_ref[...], preferred_element_type=jnp.float32)
    rs = jnp.sum(adj, axis=1, keepdims=True)
    pooled = acc / rs
    cat = jnp.concatenate([hblk_ref[...], pooled], axis=1)
    out_ref[...] = (
        jnp.dot(cat, wt_ref[...], preferred_element_type=jnp.float32) + b_ref[...]
    )


@jax.jit
def kernel(h, adj, W_lg, b_lg):
    n, d = h.shape
    wt = W_lg.T
    b = b_lg.reshape(1, d)
    return pl.pallas_call(
        _fused_block,
        grid=(_NB,),
        in_specs=[
            pl.BlockSpec(memory_space=pl.ANY),
            pl.BlockSpec((n, d), lambda i: (0, 0)),
            pl.BlockSpec((_BM, d), lambda i: (i, 0)),
            pl.BlockSpec((2 * d, d), lambda i: (0, 0)),
            pl.BlockSpec((1, d), lambda i: (0, 0)),
        ],
        out_specs=pl.BlockSpec((_BM, d), lambda i: (i, 0)),
        out_shape=jax.ShapeDtypeStruct((n, d), jnp.float32),
        scratch_shapes=[
            pltpu.VMEM((_S, _BM, _N), jnp.float32),
            pltpu.SemaphoreType.DMA((_S,)),
        ],
    )(adj, h, h, wt, b)
